# Initial kernel scaffold; baseline (speedup 1.0000x reference)
import jax
import jax.numpy as jnp
from jax.experimental import pallas as pl


def kernel(user, games, user_table, game_table, wav_w, wav_b):
    raise NotImplementedError("write your pallas kernel here")


# speedup vs baseline: 1.9326x; 1.9326x over previous
import functools

import jax
import jax.numpy as jnp
from jax import lax
from jax.experimental import pallas as pl
from jax.experimental.pallas import tpu as pltpu
from jax.experimental.pallas import tpu_sc as plsc

NUM_CORES = 2
NUM_SUBCORES = 16
LANES = 16


def _chunk_offsets(e):
  full = e // LANES
  offs = [c * LANES for c in range(full)]
  if e % LANES:
    offs.append(e - LANES)
  return offs


@jax.jit
def kernel(user, games, user_table, game_table, wav_w, wav_b):
  s_len = games.shape[0]
  e = game_table.shape[1]
  rows_per_sub = -(-s_len // (NUM_SUBCORES * LANES)) * LANES
  slots = rows_per_sub * NUM_SUBCORES
  offs = _chunk_offsets(e)
  n_chunks = len(offs)
  e_pad = n_chunks * LANES

  w_eff = wav_w[0, :, 0].astype(jnp.float32) / jnp.float32(e)
  w_pad = jnp.zeros((slots,), jnp.float32).at[:s_len].set(w_eff)
  w_bcast = jnp.broadcast_to(w_pad[:, None], (slots, LANES))
  idx_pad = jnp.zeros((slots,), jnp.int32).at[:s_len].set(games.astype(jnp.int32))
  user_idx = jnp.full((8,), user, dtype=jnp.int32)
  bias16 = jnp.broadcast_to(wav_b.astype(jnp.float32), (LANES,))

  mesh = plsc.VectorSubcoreMesh(
      core_axis_name="c", subcore_axis_name="s",
      num_cores=NUM_CORES, num_subcores=NUM_SUBCORES)

  @functools.partial(
      pl.kernel,
      out_type=jax.ShapeDtypeStruct((1, 3 * e), jnp.float32),
      mesh=mesh,
      scratch_types=[
          pltpu.VMEM((rows_per_sub,), jnp.int32),
          pltpu.VMEM((rows_per_sub, LANES), jnp.float32),
          pltpu.VMEM((rows_per_sub, e), jnp.float32),
          pltpu.VMEM((e_pad,), jnp.float32),
          pltpu.VMEM((NUM_SUBCORES, e_pad), jnp.float32),
          pltpu.VMEM((8,), jnp.int32),
          pltpu.VMEM((e,), jnp.float32),
          pltpu.VMEM((LANES,), jnp.float32),
          pltpu.VMEM((3 * e,), jnp.float32),
          pltpu.VMEM_SHARED((NUM_SUBCORES, e_pad), jnp.float32),
          pltpu.SemaphoreType.DMA,
      ],
  )
  def sc_kernel(ut_hbm, gt_hbm, idx_hbm, wbc_hbm, uarr_hbm, bias_hbm, out_hbm,
                idx_v, w_v, rows_v, partial_v, allp_v, uidx_v, urow_v,
                bias_v, out_v, shared, sem):
    sid = lax.axis_index("s")
    base = sid * rows_per_sub

    pltpu.sync_copy(idx_hbm.at[pl.ds(base, rows_per_sub)], idx_v)
    pltpu.sync_copy(wbc_hbm.at[pl.ds(base, rows_per_sub)], w_v)
    idx_vec = idx_v[:]
    copies = [
        pltpu.async_copy(gt_hbm.at[idx_vec[s]], rows_v.at[s], sem)
        for s in range(rows_per_sub)
    ]
    for c in copies:
      c.wait()

    acc = [jnp.zeros((LANES,), jnp.float32) for _ in range(n_chunks)]
    for s in range(rows_per_sub):
      wv = w_v[s, :]
      for c in range(n_chunks):
        acc[c] = acc[c] + wv * rows_v[s, pl.ds(offs[c], LANES)]
    for c in range(n_chunks):
      partial_v[pl.ds(c * LANES, LANES)] = acc[c]

    pltpu.sync_copy(partial_v, shared.at[sid])
    plsc.subcore_barrier()

    @pl.when(sid == 0)
    def _():
      pltpu.sync_copy(uarr_hbm, uidx_v)
      uidx = uidx_v[:]
      ucopy = pltpu.async_copy(ut_hbm.at[uidx[0]], urow_v, sem)
      pltpu.sync_copy(bias_hbm, bias_v)
      pltpu.sync_copy(shared, allp_v)
      ucopy.wait()
      bias = bias_v[:]
      for c in range(n_chunks):
        wav = bias
        for t in range(NUM_SUBCORES):
          wav = wav + allp_v[t, pl.ds(c * LANES, LANES)]
        ue = urow_v[pl.ds(offs[c], LANES)]
        o = offs[c]
        out_v[pl.ds(o, LANES)] = ue
        out_v[pl.ds(o + e, LANES)] = ue * wav
        out_v[pl.ds(o + 2 * e, LANES)] = wav
      pltpu.sync_copy(out_v, out_hbm.at[0])

  return sc_kernel(user_table, game_table, idx_pad, w_bcast, user_idx, bias16)


# speedup vs baseline: 1.9372x; 1.0024x over previous
import functools

import jax
import jax.numpy as jnp
from jax import lax
from jax.experimental import pallas as pl
from jax.experimental.pallas import tpu as pltpu
from jax.experimental.pallas import tpu_sc as plsc

NUM_CORES = 2
NUM_SUBCORES = 16
LANES = 16


def _chunk_offsets(e):
  full = e // LANES
  offs = [c * LANES for c in range(full)]
  if e % LANES:
    offs.append(e - LANES)
  return offs


@jax.jit
def kernel(user, games, user_table, game_table, wav_w, wav_b):
  s_len = games.shape[0]
  e = game_table.shape[1]
  rows_per_sub = -(-s_len // (NUM_SUBCORES * LANES)) * LANES
  slots = rows_per_sub * NUM_SUBCORES
  offs = _chunk_offsets(e)
  n_chunks = len(offs)
  e_pad = n_chunks * LANES

  w_eff = wav_w[0, :, 0].astype(jnp.float32) / jnp.float32(e)
  w_pad = jnp.zeros((slots,), jnp.float32).at[:s_len].set(w_eff)
  w_bcast = jnp.broadcast_to(w_pad[:, None], (slots, LANES))
  idx_pad = jnp.zeros((slots,), jnp.int32).at[:s_len].set(games.astype(jnp.int32))
  user_idx = jnp.full((8,), user, dtype=jnp.int32)
  bias16 = jnp.broadcast_to(wav_b.astype(jnp.float32), (LANES,))

  mesh = plsc.VectorSubcoreMesh(
      core_axis_name="c", subcore_axis_name="s",
      num_cores=NUM_CORES, num_subcores=NUM_SUBCORES)

  @functools.partial(
      pl.kernel,
      out_type=jax.ShapeDtypeStruct((1, 3 * e), jnp.float32),
      mesh=mesh,
      compiler_params=pltpu.CompilerParams(use_tc_tiling_on_sc=True),
      scratch_types=[
          pltpu.VMEM((rows_per_sub,), jnp.int32),
          pltpu.VMEM((rows_per_sub, LANES), jnp.float32),
          pltpu.VMEM((rows_per_sub, e), jnp.float32),
          pltpu.VMEM((e_pad,), jnp.float32),
          pltpu.VMEM((NUM_SUBCORES, e_pad), jnp.float32),
          pltpu.VMEM((8,), jnp.int32),
          pltpu.VMEM((e,), jnp.float32),
          pltpu.VMEM((LANES,), jnp.float32),
          pltpu.VMEM((3 * e,), jnp.float32),
          pltpu.VMEM_SHARED((NUM_SUBCORES, e_pad), jnp.float32),
          pltpu.SemaphoreType.DMA,
      ],
  )
  def sc_kernel(ut_hbm, gt_hbm, idx_hbm, wbc_hbm, uarr_hbm, bias_hbm, out_hbm,
                idx_v, w_v, rows_v, partial_v, allp_v, uidx_v, urow_v,
                bias_v, out_v, shared, sem):
    sid = lax.axis_index("s")
    base = sid * rows_per_sub

    pltpu.sync_copy(idx_hbm.at[pl.ds(base, rows_per_sub)], idx_v)
    pltpu.sync_copy(wbc_hbm.at[pl.ds(base, rows_per_sub)], w_v)
    idx_vec = idx_v[:]
    copies = [
        pltpu.async_copy(gt_hbm.at[idx_vec[s]], rows_v.at[s], sem)
        for s in range(rows_per_sub)
    ]
    for c in copies:
      c.wait()

    acc = [jnp.zeros((LANES,), jnp.float32) for _ in range(n_chunks)]
    for s in range(rows_per_sub):
      wv = w_v[s, :]
      for c in range(n_chunks):
        acc[c] = acc[c] + wv * rows_v[s, pl.ds(offs[c], LANES)]
    for c in range(n_chunks):
      partial_v[pl.ds(c * LANES, LANES)] = acc[c]

    pltpu.sync_copy(partial_v, shared.at[sid])
    plsc.subcore_barrier()

    @pl.when(sid == 0)
    def _():
      pltpu.sync_copy(uarr_hbm, uidx_v)
      uidx = uidx_v[:]
      ucopy = pltpu.async_copy(ut_hbm.at[uidx[0]], urow_v, sem)
      pltpu.sync_copy(bias_hbm, bias_v)
      pltpu.sync_copy(shared, allp_v)
      ucopy.wait()
      bias = bias_v[:]
      for c in range(n_chunks):
        wav = bias
        for t in range(NUM_SUBCORES):
          wav = wav + allp_v[t, pl.ds(c * LANES, LANES)]
        ue = urow_v[pl.ds(offs[c], LANES)]
        o = offs[c]
        out_v[pl.ds(o, LANES)] = ue
        out_v[pl.ds(o + e, LANES)] = ue * wav
        out_v[pl.ds(o + 2 * e, LANES)] = wav
      pltpu.sync_copy(out_v, out_hbm.at[0])

  return sc_kernel(user_table, game_table, idx_pad, w_bcast, user_idx, bias16)


# speedup vs baseline: 35.0411x; 18.0886x over previous
import functools

import jax
import jax.numpy as jnp
from jax import lax
from jax.experimental import pallas as pl
from jax.experimental.pallas import tpu as pltpu
from jax.experimental.pallas import tpu_sc as plsc

NUM_CORES = 2
NUM_SUBCORES = 16
LANES = 16
COL_TILE = 128


def _chunk_bases(e):
  full = e // LANES
  bases = [c * LANES for c in range(full)]
  if e % LANES:
    bases.append(e - LANES)
  return bases


@jax.jit
def kernel(user, games, user_table, game_table, wav_w, wav_b):
  s_len = games.shape[0]
  e = game_table.shape[1]
  rows_per_sub = -(-s_len // (NUM_SUBCORES * LANES)) * LANES
  slots = rows_per_sub * NUM_SUBCORES
  bases = _chunk_bases(e)
  n_chunks = len(bases)
  e_pad = n_chunks * LANES

  w_eff = wav_w[0, :, 0].astype(jnp.float32) / jnp.float32(e)
  w_pad = jnp.zeros((slots,), jnp.float32).at[:s_len].set(w_eff)
  w_bcast = jnp.broadcast_to(w_pad[:, None], (slots, LANES))
  idx_pad = jnp.zeros((slots,), jnp.int32).at[:s_len].set(games.astype(jnp.int32))
  user_idx = jnp.full((LANES,), user, dtype=jnp.int32)
  bias16 = jnp.broadcast_to(wav_b.astype(jnp.float32), (LANES,))

  mesh = plsc.VectorSubcoreMesh(
      core_axis_name="c", subcore_axis_name="s",
      num_cores=NUM_CORES, num_subcores=NUM_SUBCORES)

  @functools.partial(
      pl.kernel,
      out_type=jax.ShapeDtypeStruct((1, 3 * e), jnp.float32),
      mesh=mesh,
      compiler_params=pltpu.CompilerParams(use_tc_tiling_on_sc=True,
                                           needs_layout_passes=False),
      scratch_types=[
          pltpu.VMEM((rows_per_sub,), jnp.int32),
          pltpu.VMEM((rows_per_sub, LANES), jnp.float32),
          pltpu.VMEM((e, COL_TILE), jnp.float32),
          pltpu.VMEM((e, COL_TILE), jnp.float32),
          pltpu.VMEM((e_pad,), jnp.float32),
          pltpu.VMEM((NUM_SUBCORES, e_pad), jnp.float32),
          pltpu.VMEM((LANES,), jnp.int32),
          pltpu.VMEM((LANES,), jnp.float32),
          pltpu.VMEM((3 * e,), jnp.float32),
          pltpu.VMEM_SHARED((NUM_SUBCORES, e_pad), jnp.float32),
          pltpu.SemaphoreType.DMA,
          pltpu.SemaphoreType.DMA,
      ],
  )
  def sc_kernel(ut_hbm, gt_hbm, idx_hbm, wbc_hbm, uarr_hbm, bias_hbm, out_hbm,
                idx_v, w_v, blk0, blk1, partial_v, allp_v, uidx_v,
                bias_v, out_v, shared, sem0, sem1):
    sid = lax.axis_index("s")
    base = sid * rows_per_sub
    iota = lax.iota(jnp.int32, LANES)
    blks = (blk0, blk1)
    sems = (sem0, sem1)

    pltpu.sync_copy(idx_hbm.at[pl.ds(base, rows_per_sub)], idx_v)
    pltpu.sync_copy(wbc_hbm.at[pl.ds(base, rows_per_sub)], w_v)
    idx_vec = idx_v[:]

    def start_fetch(s):
      i = idx_vec[s]
      tile_base = (i // COL_TILE) * COL_TILE
      b = s % 2
      return (pltpu.async_copy(gt_hbm.at[:, pl.ds(tile_base, COL_TILE)],
                               blks[b], sems[b]),
              i - tile_base)

    acc = [jnp.zeros((LANES,), jnp.float32) for _ in range(n_chunks)]
    pending = start_fetch(0)
    for s in range(rows_per_sub):
      dma, c0 = pending
      dma.wait()
      if s + 1 < rows_per_sub:
        nxt = start_fetch(s + 1)
      wv = w_v[s, :]
      cols = jnp.broadcast_to(c0, (LANES,))
      blk = blks[s % 2]
      for c in range(n_chunks):
        v = plsc.load_gather(blk, [iota + bases[c], cols])
        acc[c] = acc[c] + wv * v
      if s + 1 < rows_per_sub:
        pending = nxt
    for c in range(n_chunks):
      partial_v[pl.ds(c * LANES, LANES)] = acc[c]

    pltpu.sync_copy(partial_v, shared.at[sid])
    plsc.subcore_barrier()

    @pl.when(sid == 0)
    def _():
      pltpu.sync_copy(uarr_hbm, uidx_v)
      uidx = uidx_v[:]
      u = uidx[0]
      utile = (u // COL_TILE) * COL_TILE
      udma = pltpu.async_copy(ut_hbm.at[:, pl.ds(utile, COL_TILE)], blk0, sem0)
      pltpu.sync_copy(bias_hbm, bias_v)
      pltpu.sync_copy(shared, allp_v)
      udma.wait()
      ucol = jnp.broadcast_to(u - utile, (LANES,))
      bias = bias_v[:]
      for c in range(n_chunks):
        wav = bias
        for t in range(NUM_SUBCORES):
          wav = wav + allp_v[t, pl.ds(c * LANES, LANES)]
        ue = plsc.load_gather(blk0, [iota + bases[c], ucol])
        o = bases[c]
        out_v[pl.ds(o, LANES)] = ue
        out_v[pl.ds(o + e, LANES)] = ue * wav
        out_v[pl.ds(o + 2 * e, LANES)] = wav
      pltpu.sync_copy(out_v, out_hbm.at[0])

  return sc_kernel(user_table.T, game_table.T, idx_pad, w_bcast,
                   user_idx, bias16)
